# Initial kernel scaffold; baseline (speedup 1.0000x reference)
import jax
import jax.numpy as jnp
from jax.experimental import pallas as pl


def kernel(h, x, edge_index, edge_attr, W_e1, b_e1, W_e2, b_e2, W_n1, b_n1, W_n2, b_n2, W_c1, b_c1, W_c2, b_c2):
    raise NotImplementedError("write your pallas kernel here")


# speedup vs baseline: 2.4059x; 2.4059x over previous
import dataclasses
import functools

import jax
import jax.numpy as jnp
from jax import lax
from jax.experimental import pallas as pl
from jax.experimental.pallas import tpu as pltpu
from jax.experimental.pallas import tpu_sc as plsc

N = 10000
E = 320000
D = 128
XP = 16
NC = 2
NS = 16
NW = NC * NS
EP = E // NW
C = 80
NCHUNK = EP // C
NP = 10240
NSTRIPE = NP // NS

def _sc_params():
    cp = pltpu.CompilerParams()
    if "needs_layout_passes" in pltpu.CompilerParams.__dataclass_fields__:
        cp = dataclasses.replace(cp, needs_layout_passes=False)
    return cp


def _dg(v, idx16):
    return lax.gather(
        v, idx16[:, None],
        lax.GatherDimensionNumbers(offset_dims=(), collapsed_slice_dims=(0,),
                                   start_index_map=(0,)),
        (1,), mode=lax.GatherScatterMode.PROMISE_IN_BOUNDS)


_mesh_cache = []


def _vector_mesh():
    if not _mesh_cache:
        _mesh_cache.append(
            plsc.VectorSubcoreMesh(core_axis_name="c", subcore_axis_name="s"))
    return _mesh_cache[0]


def _pq_body(h_ref, wab_ref, x128_ref, p_ref, q_ref, xn_ref):
    pq = jnp.dot(h_ref[...], wab_ref[...], preferred_element_type=jnp.float32)
    p_ref[...] = pq[:, :D]
    q_ref[...] = pq[:, D:]
    xn_ref[...] = -x128_ref[...]


def _pq(h, wab, x128):
    bn = 2000
    grid = (N // bn,)
    return pl.pallas_call(
        _pq_body,
        grid=grid,
        in_specs=[
            pl.BlockSpec((bn, D), lambda i: (i, 0)),
            pl.BlockSpec((D, 2 * D), lambda i: (0, 0)),
            pl.BlockSpec((bn, D), lambda i: (i, 0)),
        ],
        out_specs=[
            pl.BlockSpec((bn, D), lambda i: (i, 0)),
            pl.BlockSpec((bn, D), lambda i: (i, 0)),
            pl.BlockSpec((bn, D), lambda i: (i, 0)),
        ],
        out_shape=[
            jax.ShapeDtypeStruct((N, D), jnp.float32),
            jax.ShapeDtypeStruct((N, D), jnp.float32),
            jax.ShapeDtypeStruct((N, D), jnp.float32),
        ],
    )(h, wab, x128)


def _gather_body(p_hbm, q_hbm, x_hbm, xn_hbm, row_hbm, col_hbm, seq_hbm,
                 g_hbm, dx_hbm,
                 idr, idc, sqg, pbuf, qbuf, xrbuf, xcbuf, g_sh, dx_sh, sem):
    cid = lax.axis_index("c")
    sid = lax.axis_index("s")
    base = (sid * NC + cid) * EP
    sbase = sid * C
    pltpu.sync_copy(seq_hbm.at[pl.ds(sbase, C)], sqg)

    @pl.loop(0, NCHUNK)
    def _(k):
        off = base + k * C
        pltpu.sync_copy(row_hbm.at[pl.ds(off, C)], idr)
        pltpu.sync_copy(col_hbm.at[pl.ds(off, C)], idc)
        pltpu.async_copy(p_hbm.at[idr], pbuf, sem).wait()
        pltpu.sync_copy(pbuf, g_sh.at[pl.ds(sbase, C)])
        pltpu.async_copy(q_hbm.at[idc], qbuf, sem).wait()
        pltpu.sync_copy(qbuf, g_sh.at[sqg], add=True)
        pltpu.sync_copy(g_sh.at[pl.ds(sbase, C)], g_hbm.at[pl.ds(off, C)])
        pltpu.async_copy(x_hbm.at[idr], xrbuf, sem).wait()
        pltpu.sync_copy(xrbuf, dx_sh.at[pl.ds(sbase, C)])
        pltpu.async_copy(xn_hbm.at[idc], xcbuf, sem).wait()
        pltpu.sync_copy(xcbuf, dx_sh.at[sqg], add=True)
        pltpu.sync_copy(dx_sh.at[pl.ds(sbase, C)], dx_hbm.at[pl.ds(off, C)])


def _gather(p, q, x128, xneg, row, col, seq):
    f = pl.kernel(
        _gather_body,
        out_type=[
            jax.ShapeDtypeStruct((E, D), jnp.float32),
            jax.ShapeDtypeStruct((E, D), jnp.float32),
        ],
        mesh=_vector_mesh(),
        scratch_types=[
            pltpu.VMEM((C,), jnp.int32),
            pltpu.VMEM((C,), jnp.int32),
            pltpu.VMEM((C,), jnp.int32),
            pltpu.VMEM((C, D), jnp.float32),
            pltpu.VMEM((C, D), jnp.float32),
            pltpu.VMEM((C, D), jnp.float32),
            pltpu.VMEM((C, D), jnp.float32),
            pltpu.VMEM_SHARED((NS * C, D), jnp.float32),
            pltpu.VMEM_SHARED((NS * C, D), jnp.float32),
            pltpu.SemaphoreType.DMA,
        ],
    )
    return f(p, q, x128, xneg, row, col, seq)


def _edge_body(g_ref, dx_ref, ea_ref, wea_ref, wd_ref, be1_ref,
               we2_ref, be2_ref, wc1_ref, bc1_ref, wc2_ref, bc2_ref,
               m_ref, wdx_ref):
    dx = dx_ref[:, :XP]
    dist = jnp.sqrt(jnp.sum(dx * dx, axis=1, keepdims=True))
    z1 = (g_ref[...]
          + jnp.dot(ea_ref[...], wea_ref[...], preferred_element_type=jnp.float32)
          + dist * wd_ref[...]
          + be1_ref[...])
    a1 = z1 * jax.nn.sigmoid(z1)
    m = jnp.dot(a1, we2_ref[...], preferred_element_type=jnp.float32) + be2_ref[...]
    m_ref[...] = m
    z2 = jnp.dot(m, wc1_ref[...], preferred_element_type=jnp.float32) + bc1_ref[...]
    t = z2 * jax.nn.sigmoid(z2)
    cw = jnp.sum(t * wc2_ref[...], axis=1, keepdims=True) + bc2_ref[...]
    wdx_ref[...] = dx_ref[:, :4] * cw


def _edge(g, dx, ea, wea, wd, be1, we2, be2, wc1, bc1, wc2, bc2):
    be = 2000
    grid = (E // be,)
    full = lambda i: (0, 0)
    return pl.pallas_call(
        _edge_body,
        grid=grid,
        in_specs=[
            pl.BlockSpec((be, D), lambda i: (i, 0)),
            pl.BlockSpec((be, D), lambda i: (i, 0)),
            pl.BlockSpec((be, 16), lambda i: (i, 0)),
            pl.BlockSpec((16, D), full),
            pl.BlockSpec((1, D), full),
            pl.BlockSpec((1, D), full),
            pl.BlockSpec((D, D), full),
            pl.BlockSpec((1, D), full),
            pl.BlockSpec((D, D), full),
            pl.BlockSpec((1, D), full),
            pl.BlockSpec((1, D), full),
            pl.BlockSpec((1, 1), full),
        ],
        out_specs=[
            pl.BlockSpec((be, D), lambda i: (i, 0)),
            pl.BlockSpec((be, 4), lambda i: (i, 0)),
        ],
        out_shape=[
            jax.ShapeDtypeStruct((E, D), jnp.float32),
            jax.ShapeDtypeStruct((E, 4), jnp.float32),
        ],
    )(g, dx, ea, wea, wd, be1, we2, be2, wc1, bc1, wc2, bc2)


def _scatter_body(m_hbm, row_hbm, z128_hbm,
                  mp_hbm,
                  m_sh, idx, mbuf):
    cid = lax.axis_index("c")
    sid = lax.axis_index("s")
    base = (sid * NC + cid) * EP
    stripe = sid * NSTRIPE
    pltpu.sync_copy(z128_hbm, m_sh.at[pl.ds(stripe, NSTRIPE)])
    plsc.subcore_barrier()

    @pl.loop(0, NCHUNK)
    def _(k):
        off = base + k * C
        pltpu.sync_copy(row_hbm.at[pl.ds(off, C)], idx)
        pltpu.sync_copy(m_hbm.at[pl.ds(off, C)], mbuf)
        pltpu.sync_copy(mbuf, m_sh.at[idx], add=True)

    plsc.subcore_barrier()
    pltpu.sync_copy(m_sh.at[pl.ds(stripe, NSTRIPE)],
                    mp_hbm.at[cid, pl.ds(stripe, NSTRIPE)])


def _scatter(m_ij, row, z128):
    f = pl.kernel(
        _scatter_body,
        out_type=jax.ShapeDtypeStruct((NC, NP, D), jnp.float32),
        mesh=_vector_mesh(),
        scratch_types=[
            pltpu.VMEM_SHARED((NP, D), jnp.float32),
            pltpu.VMEM((C,), jnp.int32),
            pltpu.VMEM((C, D), jnp.float32),
        ],
    )
    return f(m_ij, row, z128)


def _cscatter_body(w4_hbm, row_hbm, cp_hbm, idx, wvm, cacc):
    cid = lax.axis_index("c")
    sid = lax.axis_index("s")
    wid = sid * NC + cid
    base = wid * EP
    zero16 = jnp.zeros((16,), jnp.float32)

    @pl.loop(0, (N * 4) // 16)
    def _(i):
        cacc[pl.ds(i * 16, 16)] = zero16

    iota16 = lax.iota(jnp.int32, 16)
    lane = iota16 & 3
    mask4 = iota16 < 4
    vpats = [u * 4 + lane for u in range(4)]

    @pl.loop(0, NCHUNK)
    def _(k):
        off = base + k * C
        pltpu.sync_copy(row_hbm.at[pl.ds(off, C)], idx)
        pltpu.sync_copy(w4_hbm.at[pl.ds(off * 4, C * 4)], wvm)

        @pl.loop(0, C // 16)
        def _(g):
            rows16 = idx[pl.ds(g * 16, 16)]
            for t in range(4):
                vload = wvm[pl.ds((g * 4 + t) * 16, 16)]
                for u in range(4):
                    j = t * 4 + u
                    r = _dg(rows16, jnp.full((16,), j, jnp.int32))
                    v = _dg(vload, vpats[u])
                    plsc.addupdate_scatter(cacc, [r * 4 + lane], v, mask=mask4)

    pltpu.sync_copy(cacc, cp_hbm.at[wid])


def _cscatter(w4flat, row):
    f = pl.kernel(
        _cscatter_body,
        out_type=jax.ShapeDtypeStruct((NW, N * 4), jnp.float32),
        mesh=_vector_mesh(),
        compiler_params=_sc_params(),
        scratch_types=[
            pltpu.VMEM((C,), jnp.int32),
            pltpu.VMEM((C * 4,), jnp.float32),
            pltpu.VMEM((N * 4,), jnp.float32),
        ],
    )
    return f(w4flat, row)


def _node_body(h_ref, m0_ref, m1_ref, a_ref, b_ref, bn1_ref, wn2_ref, bn2_ref,
               hn_ref):
    h = h_ref[...]
    mi = m0_ref[...] + m1_ref[...]
    z = (jnp.dot(h, a_ref[...], preferred_element_type=jnp.float32)
         + jnp.dot(mi, b_ref[...], preferred_element_type=jnp.float32)
         + bn1_ref[...])
    u = z * jax.nn.sigmoid(z)
    hn_ref[...] = h + jnp.dot(u, wn2_ref[...], preferred_element_type=jnp.float32) + bn2_ref[...]


def _node(h, m0, m1, a, b, bn1, wn2, bn2):
    bn = 2000
    grid = (N // bn,)
    full = lambda i: (0, 0)
    return pl.pallas_call(
        _node_body,
        grid=grid,
        in_specs=[
            pl.BlockSpec((bn, D), lambda i: (i, 0)),
            pl.BlockSpec((bn, D), lambda i: (i, 0)),
            pl.BlockSpec((bn, D), lambda i: (i, 0)),
            pl.BlockSpec((D, D), full),
            pl.BlockSpec((D, D), full),
            pl.BlockSpec((1, D), full),
            pl.BlockSpec((D, D), full),
            pl.BlockSpec((1, D), full),
        ],
        out_specs=pl.BlockSpec((bn, D), lambda i: (i, 0)),
        out_shape=jax.ShapeDtypeStruct((N, D), jnp.float32),
    )(h, m0, m1, a, b, bn1, wn2, bn2)


def _coord_body(x4_ref, cp_ref, xn_ref):
    xn_ref[...] = x4_ref[...] + jnp.sum(cp_ref[...], axis=0)


def _coord(x4flat, cp):
    return pl.pallas_call(
        _coord_body,
        grid=(1,),
        in_specs=[
            pl.BlockSpec((N * 4,), lambda i: (0,)),
            pl.BlockSpec((NW, N * 4), lambda i: (0, 0)),
        ],
        out_specs=pl.BlockSpec((N * 4,), lambda i: (0,)),
        out_shape=jax.ShapeDtypeStruct((N * 4,), jnp.float32),
    )(x4flat, cp)


def kernel(h, x, edge_index, edge_attr,
           W_e1, b_e1, W_e2, b_e2,
           W_n1, b_n1, W_n2, b_n2,
           W_c1, b_c1, W_c2, b_c2):
    row = edge_index[0]
    col = edge_index[1]
    x128 = jnp.pad(x, ((0, 0), (0, D - 3)))
    wab = jnp.concatenate([W_e1[:D], W_e1[D:2 * D]], axis=1)
    wea = W_e1[2 * D:2 * D + 16]
    wd = W_e1[2 * D + 16:].reshape(1, D)
    seq = jnp.arange(NS * C, dtype=jnp.int32)
    z128 = jnp.zeros((NSTRIPE, D), jnp.float32)

    p, q, xneg = _pq(h, wab, x128)
    g, dx = _gather(p, q, x128, xneg, row, col, seq)
    m_ij, wdx = _edge(
        g, dx, edge_attr, wea, wd, b_e1.reshape(1, D),
        W_e2, b_e2.reshape(1, D), W_c1, b_c1.reshape(1, D),
        W_c2.reshape(1, D), b_c2.reshape(1, 1))
    mp = _scatter(m_ij, row, z128)
    cp = _cscatter(wdx.reshape(E * 4), row)
    hn = _node(
        h, mp[0, :N], mp[1, :N], W_n1[:D], W_n1[D:], b_n1.reshape(1, D),
        W_n2, b_n2.reshape(1, D))
    xn = _coord(jnp.pad(x, ((0, 0), (0, 1))).reshape(N * 4), cp)
    return (hn, xn.reshape(N, 4)[:, :3])


# speedup vs baseline: 2.9993x; 1.2467x over previous
import dataclasses
import functools

import jax
import jax.numpy as jnp
from jax import lax
from jax.experimental import pallas as pl
from jax.experimental.pallas import tpu as pltpu
from jax.experimental.pallas import tpu_sc as plsc

N = 10000
E = 320000
D = 128
XP = 16
NC = 2
NS = 16
NW = NC * NS
EP = E // NW
C = 80
NCHUNK = EP // C
NP = 10240
NSTRIPE = NP // NS

def _sc_params():
    cp = pltpu.CompilerParams()
    if "needs_layout_passes" in pltpu.CompilerParams.__dataclass_fields__:
        cp = dataclasses.replace(cp, needs_layout_passes=False)
    return cp


def _dg(v, idx16):
    return lax.gather(
        v, idx16[:, None],
        lax.GatherDimensionNumbers(offset_dims=(), collapsed_slice_dims=(0,),
                                   start_index_map=(0,)),
        (1,), mode=lax.GatherScatterMode.PROMISE_IN_BOUNDS)


_mesh_cache = []


def _vector_mesh():
    if not _mesh_cache:
        _mesh_cache.append(
            plsc.VectorSubcoreMesh(core_axis_name="c", subcore_axis_name="s"))
    return _mesh_cache[0]


def _pq_body(h_ref, wab_ref, p_ref, q_ref):
    pq = jnp.dot(h_ref[...], wab_ref[...], preferred_element_type=jnp.float32)
    p_ref[...] = pq[:, :D]
    q_ref[...] = pq[:, D:]


def _pq(h, wab):
    bn = 2000
    grid = (N // bn,)
    return pl.pallas_call(
        _pq_body,
        grid=grid,
        in_specs=[
            pl.BlockSpec((bn, D), lambda i: (i, 0)),
            pl.BlockSpec((D, 2 * D), lambda i: (0, 0)),
        ],
        out_specs=[
            pl.BlockSpec((bn, D), lambda i: (i, 0)),
            pl.BlockSpec((bn, D), lambda i: (i, 0)),
        ],
        out_shape=[
            jax.ShapeDtypeStruct((N, D), jnp.float32),
            jax.ShapeDtypeStruct((N, D), jnp.float32),
        ],
    )(h, wab)


def _gather_body(p_hbm, q_hbm, x4_hbm, row_hbm, col_hbm, seq_hbm,
                 g_hbm, dx_hbm,
                 idr, idc, sqg, pbuf, qbuf, x4v, dxbuf, g_sh, sem):
    cid = lax.axis_index("c")
    sid = lax.axis_index("s")
    base = (sid * NC + cid) * EP
    sbase = sid * C
    pltpu.sync_copy(seq_hbm.at[pl.ds(sbase, C)], sqg)
    pltpu.sync_copy(x4_hbm, x4v)
    iota16 = lax.iota(jnp.int32, 16)
    lane = iota16 & 3
    pats = [u * 4 + (iota16 >> 2) for u in range(4)]

    @pl.loop(0, NCHUNK)
    def _(k):
        off = base + k * C
        pltpu.sync_copy(row_hbm.at[pl.ds(off, C)], idr)
        pltpu.sync_copy(col_hbm.at[pl.ds(off, C)], idc)
        cp1 = pltpu.async_copy(p_hbm.at[idr], pbuf, sem)
        cp2 = pltpu.async_copy(q_hbm.at[idc], qbuf, sem)

        @pl.loop(0, C // 16)
        def _(g):
            r16 = idr[pl.ds(g * 16, 16)]
            c16 = idc[pl.ds(g * 16, 16)]
            for u in range(4):
                pos_r = _dg(r16, pats[u]) * 4 + lane
                pos_c = _dg(c16, pats[u]) * 4 + lane
                xr = plsc.load_gather(x4v, [pos_r])
                xc = plsc.load_gather(x4v, [pos_c])
                dxbuf[pl.ds((g * 4 + u) * 16, 16)] = xr - xc

        pltpu.sync_copy(dxbuf, dx_hbm.at[pl.ds(off * 4, C * 4)])
        cp1.wait()
        pltpu.sync_copy(pbuf, g_sh.at[pl.ds(sbase, C)])
        cp2.wait()
        pltpu.sync_copy(qbuf, g_sh.at[sqg], add=True)
        pltpu.sync_copy(g_sh.at[pl.ds(sbase, C)], g_hbm.at[pl.ds(off, C)])


def _gather(p, q, x4flat, row, col, seq):
    f = pl.kernel(
        _gather_body,
        out_type=[
            jax.ShapeDtypeStruct((E, D), jnp.float32),
            jax.ShapeDtypeStruct((E * 4,), jnp.float32),
        ],
        mesh=_vector_mesh(),
        compiler_params=_sc_params(),
        scratch_types=[
            pltpu.VMEM((C,), jnp.int32),
            pltpu.VMEM((C,), jnp.int32),
            pltpu.VMEM((C,), jnp.int32),
            pltpu.VMEM((C, D), jnp.float32),
            pltpu.VMEM((C, D), jnp.float32),
            pltpu.VMEM((N * 4,), jnp.float32),
            pltpu.VMEM((C * 4,), jnp.float32),
            pltpu.VMEM_SHARED((NS * C, D), jnp.float32),
            pltpu.SemaphoreType.DMA,
        ],
    )
    return f(p, q, x4flat, row, col, seq)


def _edge_body(g_ref, dx_ref, ea_ref, wea_ref, wd_ref, be1_ref,
               we2_ref, be2_ref, wc1_ref, bc1_ref, wc2_ref, bc2_ref,
               m_ref, wdx_ref):
    dx = dx_ref[...]
    dist = jnp.sqrt(jnp.sum(dx * dx, axis=1, keepdims=True))
    z1 = (g_ref[...]
          + jnp.dot(ea_ref[...], wea_ref[...], preferred_element_type=jnp.float32)
          + dist * wd_ref[...]
          + be1_ref[...])
    a1 = z1 * jax.nn.sigmoid(z1)
    m = jnp.dot(a1, we2_ref[...], preferred_element_type=jnp.float32) + be2_ref[...]
    m_ref[...] = m
    z2 = jnp.dot(m, wc1_ref[...], preferred_element_type=jnp.float32) + bc1_ref[...]
    t = z2 * jax.nn.sigmoid(z2)
    cw = jnp.sum(t * wc2_ref[...], axis=1, keepdims=True) + bc2_ref[...]
    wdx_ref[...] = dx * cw


def _edge(g, dx, ea, wea, wd, be1, we2, be2, wc1, bc1, wc2, bc2):
    be = 2000
    grid = (E // be,)
    full = lambda i: (0, 0)
    return pl.pallas_call(
        _edge_body,
        grid=grid,
        in_specs=[
            pl.BlockSpec((be, D), lambda i: (i, 0)),
            pl.BlockSpec((be, 4), lambda i: (i, 0)),
            pl.BlockSpec((be, 16), lambda i: (i, 0)),
            pl.BlockSpec((16, D), full),
            pl.BlockSpec((1, D), full),
            pl.BlockSpec((1, D), full),
            pl.BlockSpec((D, D), full),
            pl.BlockSpec((1, D), full),
            pl.BlockSpec((D, D), full),
            pl.BlockSpec((1, D), full),
            pl.BlockSpec((1, D), full),
            pl.BlockSpec((1, 1), full),
        ],
        out_specs=[
            pl.BlockSpec((be, D), lambda i: (i, 0)),
            pl.BlockSpec((be, 4), lambda i: (i, 0)),
        ],
        out_shape=[
            jax.ShapeDtypeStruct((E, D), jnp.float32),
            jax.ShapeDtypeStruct((E, 4), jnp.float32),
        ],
    )(g, dx, ea, wea, wd, be1, we2, be2, wc1, bc1, wc2, bc2)


def _scatter_body(m_hbm, row_hbm, z128_hbm,
                  mp_hbm,
                  m_sh, idx, mbuf):
    cid = lax.axis_index("c")
    sid = lax.axis_index("s")
    base = (sid * NC + cid) * EP
    stripe = sid * NSTRIPE
    pltpu.sync_copy(z128_hbm, m_sh.at[pl.ds(stripe, NSTRIPE)])
    plsc.subcore_barrier()

    @pl.loop(0, NCHUNK)
    def _(k):
        off = base + k * C
        pltpu.sync_copy(row_hbm.at[pl.ds(off, C)], idx)
        pltpu.sync_copy(m_hbm.at[pl.ds(off, C)], mbuf)
        pltpu.sync_copy(mbuf, m_sh.at[idx], add=True)

    plsc.subcore_barrier()
    pltpu.sync_copy(m_sh.at[pl.ds(stripe, NSTRIPE)],
                    mp_hbm.at[cid, pl.ds(stripe, NSTRIPE)])


def _scatter(m_ij, row, z128):
    f = pl.kernel(
        _scatter_body,
        out_type=jax.ShapeDtypeStruct((NC, NP, D), jnp.float32),
        mesh=_vector_mesh(),
        scratch_types=[
            pltpu.VMEM_SHARED((NP, D), jnp.float32),
            pltpu.VMEM((C,), jnp.int32),
            pltpu.VMEM((C, D), jnp.float32),
        ],
    )
    return f(m_ij, row, z128)


def _cscatter_body(w4_hbm, row_hbm, cp_hbm, idx, wvm, cacc):
    cid = lax.axis_index("c")
    sid = lax.axis_index("s")
    wid = sid * NC + cid
    base = wid * EP
    zero16 = jnp.zeros((16,), jnp.float32)

    @pl.loop(0, (N * 4) // 16)
    def _(i):
        cacc[pl.ds(i * 16, 16)] = zero16

    iota16 = lax.iota(jnp.int32, 16)
    lane = iota16 & 3
    mask4 = iota16 < 4
    vpats = [u * 4 + lane for u in range(4)]

    @pl.loop(0, NCHUNK)
    def _(k):
        off = base + k * C
        pltpu.sync_copy(row_hbm.at[pl.ds(off, C)], idx)
        pltpu.sync_copy(w4_hbm.at[pl.ds(off * 4, C * 4)], wvm)

        @pl.loop(0, C // 16)
        def _(g):
            rows16 = idx[pl.ds(g * 16, 16)]
            for t in range(4):
                vload = wvm[pl.ds((g * 4 + t) * 16, 16)]
                for u in range(4):
                    j = t * 4 + u
                    r = _dg(rows16, jnp.full((16,), j, jnp.int32))
                    v = _dg(vload, vpats[u])
                    plsc.addupdate_scatter(cacc, [r * 4 + lane], v, mask=mask4)

    pltpu.sync_copy(cacc, cp_hbm.at[wid])


def _cscatter(w4flat, row):
    f = pl.kernel(
        _cscatter_body,
        out_type=jax.ShapeDtypeStruct((NW, N * 4), jnp.float32),
        mesh=_vector_mesh(),
        compiler_params=_sc_params(),
        scratch_types=[
            pltpu.VMEM((C,), jnp.int32),
            pltpu.VMEM((C * 4,), jnp.float32),
            pltpu.VMEM((N * 4,), jnp.float32),
        ],
    )
    return f(w4flat, row)


def _node_body(h_ref, m0_ref, m1_ref, a_ref, b_ref, bn1_ref, wn2_ref, bn2_ref,
               hn_ref):
    h = h_ref[...]
    mi = m0_ref[...] + m1_ref[...]
    z = (jnp.dot(h, a_ref[...], preferred_element_type=jnp.float32)
         + jnp.dot(mi, b_ref[...], preferred_element_type=jnp.float32)
         + bn1_ref[...])
    u = z * jax.nn.sigmoid(z)
    hn_ref[...] = h + jnp.dot(u, wn2_ref[...], preferred_element_type=jnp.float32) + bn2_ref[...]


def _node(h, m0, m1, a, b, bn1, wn2, bn2):
    bn = 2000
    grid = (N // bn,)
    full = lambda i: (0, 0)
    return pl.pallas_call(
        _node_body,
        grid=grid,
        in_specs=[
            pl.BlockSpec((bn, D), lambda i: (i, 0)),
            pl.BlockSpec((bn, D), lambda i: (i, 0)),
            pl.BlockSpec((bn, D), lambda i: (i, 0)),
            pl.BlockSpec((D, D), full),
            pl.BlockSpec((D, D), full),
            pl.BlockSpec((1, D), full),
            pl.BlockSpec((D, D), full),
            pl.BlockSpec((1, D), full),
        ],
        out_specs=pl.BlockSpec((bn, D), lambda i: (i, 0)),
        out_shape=jax.ShapeDtypeStruct((N, D), jnp.float32),
    )(h, m0, m1, a, b, bn1, wn2, bn2)


def _coord_body(x4_ref, cp_ref, xn_ref):
    xn_ref[...] = x4_ref[...] + jnp.sum(cp_ref[...], axis=0)


def _coord(x4flat, cp):
    return pl.pallas_call(
        _coord_body,
        grid=(1,),
        in_specs=[
            pl.BlockSpec((N * 4,), lambda i: (0,)),
            pl.BlockSpec((NW, N * 4), lambda i: (0, 0)),
        ],
        out_specs=pl.BlockSpec((N * 4,), lambda i: (0,)),
        out_shape=jax.ShapeDtypeStruct((N * 4,), jnp.float32),
    )(x4flat, cp)


def kernel(h, x, edge_index, edge_attr,
           W_e1, b_e1, W_e2, b_e2,
           W_n1, b_n1, W_n2, b_n2,
           W_c1, b_c1, W_c2, b_c2):
    row = edge_index[0]
    col = edge_index[1]
    x4flat = jnp.pad(x, ((0, 0), (0, 1))).reshape(N * 4)
    wab = jnp.concatenate([W_e1[:D], W_e1[D:2 * D]], axis=1)
    wea = W_e1[2 * D:2 * D + 16]
    wd = W_e1[2 * D + 16:].reshape(1, D)
    seq = jnp.arange(NS * C, dtype=jnp.int32)
    z128 = jnp.zeros((NSTRIPE, D), jnp.float32)

    p, q = _pq(h, wab)
    g, dxflat = _gather(p, q, x4flat, row, col, seq)
    dx = dxflat.reshape(E, 4)
    m_ij, wdx = _edge(
        g, dx, edge_attr, wea, wd, b_e1.reshape(1, D),
        W_e2, b_e2.reshape(1, D), W_c1, b_c1.reshape(1, D),
        W_c2.reshape(1, D), b_c2.reshape(1, 1))
    mp = _scatter(m_ij, row, z128)
    cp = _cscatter(wdx.reshape(E * 4), row)
    hn = _node(
        h, mp[0, :N], mp[1, :N], W_n1[:D], W_n1[D:], b_n1.reshape(1, D),
        W_n2, b_n2.reshape(1, D))
    xn = _coord(x4flat, cp)
    return (hn, xn.reshape(N, 4)[:, :3])
